# Initial kernel scaffold; baseline (speedup 1.0000x reference)
import jax
import jax.numpy as jnp
from jax.experimental import pallas as pl


def kernel(inp):
    raise NotImplementedError("write your pallas kernel here")


# speedup vs baseline: 4.4784x; 4.4784x over previous
import jax
import jax.numpy as jnp
from jax.experimental import pallas as pl
from jax.experimental.pallas import tpu as pltpu

B = 4096
D = 64
NK = 10
KTH = 7
BIG = 1000000000.0
ROWS1 = 512
ROWS2 = 512
MED_CHUNK = 512


def _d2_block(rows, allx):
    gram = jax.lax.dot_general(rows, allx, (((1,), (1,)), ((), ())),
                               preferred_element_type=jnp.float32)
    rsq = jnp.sum(rows * rows, axis=1, keepdims=True)
    ones = jnp.ones((1, D), jnp.float32)
    csq = jax.lax.dot_general(ones, allx * allx, (((1,), (1,)), ((), ())),
                              preferred_element_type=jnp.float32,
                              precision=jax.lax.Precision.HIGHEST)
    return jnp.maximum(rsq + csq - 2.0 * gram, 0.0)


def _knn_kernel(inp_ref, rows_ref, kth_col_ref, kth_lane_ref, colsel_ref):
    i = pl.program_id(0)
    allx = inp_ref[...]
    rows = rows_ref[...]
    r = rows.shape[0]
    row_g = i * r + jax.lax.broadcasted_iota(jnp.int32, (r, B), 0)
    col_i = jax.lax.broadcasted_iota(jnp.int32, (r, B), 1)
    diag = row_g == col_i
    d2 = jnp.where(diag, BIG, _d2_block(rows, allx))

    d2w = d2
    kth = None
    t10 = None
    for k in range(NK):
        m = jnp.min(d2w, axis=1, keepdims=True)
        if k == KTH:
            kth = m
        if k == NK - 1:
            t10 = m
        else:
            d2w = jnp.where(d2w == m, BIG, d2w)

    sel = (d2 <= t10).astype(jnp.float32)
    hits = jnp.max(sel, axis=0, keepdims=True)
    kl = jnp.sum(jnp.where(diag, kth, 0.0), axis=0, keepdims=True)

    kth_col_ref[...] = kth

    @pl.when(i == 0)
    def _init():
        colsel_ref[...] = jnp.zeros_like(colsel_ref)
        kth_lane_ref[...] = jnp.zeros_like(kth_lane_ref)

    colsel_ref[...] = jnp.maximum(colsel_ref[...], hits)
    kth_lane_ref[...] = kth_lane_ref[...] + kl


def _aff_kernel(kth_col_ref, kth_lane_ref, colsel_ref, inp_ref, rows_ref,
                out_ref, sig_ref):
    i = pl.program_id(0)

    @pl.when(i == 0)
    def _sigma():
        x_col = kth_col_ref[...]
        x_lane = kth_lane_ref[...]
        cnt = jnp.zeros((1, B), jnp.float32)
        for k in range(B // MED_CHUNK):
            xc = x_col[k * MED_CHUNK:(k + 1) * MED_CHUNK, :]
            cnt = cnt + jnp.sum((xc <= x_lane).astype(jnp.float32),
                                axis=0, keepdims=True)
        med = jnp.min(jnp.where(cnt >= float(B // 2 + (B % 2)), x_lane, BIG))
        sig_ref[0] = jnp.sqrt(jnp.sqrt(med))

    sigma = sig_ref[0]
    inv = 1.0 / (2.0 * sigma * sigma)

    allx = inp_ref[...]
    rows = rows_ref[...]
    r = rows.shape[0]
    row_g = i * r + jax.lax.broadcasted_iota(jnp.int32, (r, B), 0)
    col_i = jax.lax.broadcasted_iota(jnp.int32, (r, B), 1)
    diag = row_g == col_i
    dist = jnp.where(diag, BIG, jnp.sqrt(_d2_block(rows, allx)))
    e = jnp.exp(-dist * inv)

    cs_lane = colsel_ref[...]
    cs_row = jnp.sum(jnp.where(diag, cs_lane, 0.0), axis=1, keepdims=True)
    out_ref[...] = e * ((cs_row + cs_lane) * 0.5)


def kernel(inp):
    n1 = B // ROWS1
    kth_col, kth_lane, colsel = pl.pallas_call(
        _knn_kernel,
        grid=(n1,),
        in_specs=[
            pl.BlockSpec((B, D), lambda i: (0, 0)),
            pl.BlockSpec((ROWS1, D), lambda i: (i, 0)),
        ],
        out_specs=[
            pl.BlockSpec((ROWS1, 1), lambda i: (i, 0)),
            pl.BlockSpec((1, B), lambda i: (0, 0)),
            pl.BlockSpec((1, B), lambda i: (0, 0)),
        ],
        out_shape=[
            jax.ShapeDtypeStruct((B, 1), jnp.float32),
            jax.ShapeDtypeStruct((1, B), jnp.float32),
            jax.ShapeDtypeStruct((1, B), jnp.float32),
        ],
        compiler_params=pltpu.CompilerParams(
            dimension_semantics=("arbitrary",)),
    )(inp, inp)

    n2 = B // ROWS2
    sym = pl.pallas_call(
        _aff_kernel,
        grid=(n2,),
        in_specs=[
            pl.BlockSpec((B, 1), lambda i: (0, 0)),
            pl.BlockSpec((1, B), lambda i: (0, 0)),
            pl.BlockSpec((1, B), lambda i: (0, 0)),
            pl.BlockSpec((B, D), lambda i: (0, 0)),
            pl.BlockSpec((ROWS2, D), lambda i: (i, 0)),
        ],
        out_specs=pl.BlockSpec((ROWS2, B), lambda i: (i, 0)),
        out_shape=jax.ShapeDtypeStruct((B, B), jnp.float32),
        scratch_shapes=[pltpu.SMEM((1,), jnp.float32)],
        compiler_params=pltpu.CompilerParams(
            dimension_semantics=("arbitrary",)),
    )(kth_col, kth_lane, colsel, inp, inp)
    return sym


# speedup vs baseline: 4.9382x; 1.1027x over previous
import jax
import jax.numpy as jnp
from jax.experimental import pallas as pl
from jax.experimental.pallas import tpu as pltpu

B = 4096
D = 64
NK = 10
KTH = 7
BIG = 1000000000.0
ROWS1 = 512
ROWS2 = 512
MED_RANK = B // 2 + (B % 2)


def _d2_block(rows, allx):
    gram = jax.lax.dot_general(rows, allx, (((1,), (1,)), ((), ())),
                               preferred_element_type=jnp.float32)
    rsq = jnp.sum(rows * rows, axis=1, keepdims=True)
    ones = jnp.ones((1, D), jnp.float32)
    csq = jax.lax.dot_general(ones, allx * allx, (((1,), (1,)), ((), ())),
                              preferred_element_type=jnp.float32,
                              precision=jax.lax.Precision.HIGHEST)
    return jnp.maximum(rsq + csq - 2.0 * gram, 0.0)


def _knn_kernel(inp_ref, rows_ref, kth_lane_ref, colsel_ref, sigma_ref):
    i = pl.program_id(0)
    n = pl.num_programs(0)
    allx = inp_ref[...]
    rows = rows_ref[...]
    r = rows.shape[0]
    row_g = i * r + jax.lax.broadcasted_iota(jnp.int32, (r, B), 0)
    col_i = jax.lax.broadcasted_iota(jnp.int32, (r, B), 1)
    d2 = jnp.where(row_g == col_i, BIG, _d2_block(rows, allx))

    m = jnp.min(d2, axis=1, keepdims=True)
    kth = m
    for k in range(1, NK):
        m = jnp.min(jnp.where(d2 <= m, BIG, d2), axis=1, keepdims=True)
        if k == KTH:
            kth = m
    t10 = m

    hits = jnp.max((d2 <= t10).astype(jnp.float32), axis=0, keepdims=True)

    lr = jax.lax.broadcasted_iota(jnp.int32, (r, r), 0)
    lc = jax.lax.broadcasted_iota(jnp.int32, (r, r), 1)
    kl = jnp.sum(jnp.where(lr == lc, kth, 0.0), axis=0, keepdims=True)
    kth_lane_ref[:, pl.ds(i * r, r)] = kl

    @pl.when(i == 0)
    def _init():
        colsel_ref[...] = jnp.zeros_like(colsel_ref)

    colsel_ref[...] = jnp.maximum(colsel_ref[...], hits)

    @pl.when(i == n - 1)
    def _sigma():
        x = kth_lane_ref[...]

        def body(_, lohi):
            lo, hi = lohi
            mid = lo + ((hi - lo) >> 1)
            t = jax.lax.bitcast_convert_type(
                jnp.full((1, B), mid, jnp.int32), jnp.float32)
            cnt = jnp.sum((x <= t).astype(jnp.float32))
            return (jnp.where(cnt >= float(MED_RANK), lo, mid + 1),
                    jnp.where(cnt >= float(MED_RANK), mid, hi))

        lo, _ = jax.lax.fori_loop(
            0, 31, body, (jnp.int32(0), jnp.int32(0x7F800000)))
        med_d2 = jax.lax.bitcast_convert_type(lo, jnp.float32)
        sigma_ref[...] = jnp.broadcast_to(jnp.sqrt(jnp.sqrt(med_d2)), (1, 1))


def _aff_kernel(sigma_ref, colsel_ref, inp_ref, rows_ref, out_ref):
    i = pl.program_id(0)
    sigma = sigma_ref[0, 0]
    inv = 1.0 / (2.0 * sigma * sigma)

    allx = inp_ref[...]
    rows = rows_ref[...]
    r = rows.shape[0]
    row_g = i * r + jax.lax.broadcasted_iota(jnp.int32, (r, B), 0)
    col_i = jax.lax.broadcasted_iota(jnp.int32, (r, B), 1)
    diag = row_g == col_i
    dist = jnp.where(diag, BIG, jnp.sqrt(_d2_block(rows, allx)))
    e = jnp.exp(-dist * inv)

    cs_lane = colsel_ref[...]
    cs_chunk = colsel_ref[:, pl.ds(i * r, r)]
    lr = jax.lax.broadcasted_iota(jnp.int32, (r, r), 0)
    lc = jax.lax.broadcasted_iota(jnp.int32, (r, r), 1)
    cs_row = jnp.sum(jnp.where(lr == lc, cs_chunk, 0.0), axis=1,
                     keepdims=True)
    out_ref[...] = e * ((cs_row + cs_lane) * 0.5)


def kernel(inp):
    n1 = B // ROWS1
    kth_lane, colsel, sigma = pl.pallas_call(
        _knn_kernel,
        grid=(n1,),
        in_specs=[
            pl.BlockSpec((B, D), lambda i: (0, 0)),
            pl.BlockSpec((ROWS1, D), lambda i: (i, 0)),
        ],
        out_specs=[
            pl.BlockSpec((1, B), lambda i: (0, 0)),
            pl.BlockSpec((1, B), lambda i: (0, 0)),
            pl.BlockSpec((1, 1), lambda i: (0, 0)),
        ],
        out_shape=[
            jax.ShapeDtypeStruct((1, B), jnp.float32),
            jax.ShapeDtypeStruct((1, B), jnp.float32),
            jax.ShapeDtypeStruct((1, 1), jnp.float32),
        ],
        compiler_params=pltpu.CompilerParams(
            dimension_semantics=("arbitrary",)),
    )(inp, inp)

    n2 = B // ROWS2
    sym = pl.pallas_call(
        _aff_kernel,
        grid=(n2,),
        in_specs=[
            pl.BlockSpec((1, 1), lambda i: (0, 0)),
            pl.BlockSpec((1, B), lambda i: (0, 0)),
            pl.BlockSpec((B, D), lambda i: (0, 0)),
            pl.BlockSpec((ROWS2, D), lambda i: (i, 0)),
        ],
        out_specs=pl.BlockSpec((ROWS2, B), lambda i: (i, 0)),
        out_shape=jax.ShapeDtypeStruct((B, B), jnp.float32),
        compiler_params=pltpu.CompilerParams(
            dimension_semantics=("arbitrary",)),
    )(sigma, colsel, inp, inp)
    return sym


# speedup vs baseline: 5.6294x; 1.1400x over previous
import jax
import jax.numpy as jnp
from jax.experimental import pallas as pl
from jax.experimental.pallas import tpu as pltpu

B = 4096
D = 64
NK = 10
KTH = 7
BIG = 1000000000.0
ROWS1 = 512
ROWS2 = 512
MED_RANK = B // 2 + (B % 2)


def _knn_kernel(inp_ref, rows_ref, dist_ref, kth_lane_ref, colsel_ref,
                sigma_ref):
    i = pl.program_id(0)
    n = pl.num_programs(0)
    allx = inp_ref[...]
    rows = rows_ref[...]
    r = rows.shape[0]

    gram = jax.lax.dot_general(rows, allx, (((1,), (1,)), ((), ())),
                               preferred_element_type=jnp.float32)
    rsq = jnp.sum(rows * rows, axis=1, keepdims=True)
    ones = jnp.ones((1, D), jnp.float32)
    csq = jax.lax.dot_general(ones, allx * allx, (((1,), (1,)), ((), ())),
                              preferred_element_type=jnp.float32,
                              precision=jax.lax.Precision.HIGHEST)
    d2raw = jnp.maximum(rsq + csq - 2.0 * gram, 0.0)

    row_g = i * r + jax.lax.broadcasted_iota(jnp.int32, (r, B), 0)
    col_i = jax.lax.broadcasted_iota(jnp.int32, (r, B), 1)
    diag = row_g == col_i
    d2 = jnp.where(diag, BIG, d2raw)
    dist_ref[...] = jnp.where(diag, BIG, jnp.sqrt(d2raw))

    m = jnp.min(d2, axis=1, keepdims=True)
    kth = m
    for k in range(1, NK):
        m = jnp.min(jnp.where(d2 <= m, BIG, d2), axis=1, keepdims=True)
        if k == KTH:
            kth = m
    t10 = m

    hits = jnp.max(jnp.where(d2 <= t10, 0.5, 0.0), axis=0, keepdims=True)

    lr = jax.lax.broadcasted_iota(jnp.int32, (r, r), 0)
    lc = jax.lax.broadcasted_iota(jnp.int32, (r, r), 1)
    kl = jnp.sum(jnp.where(lr == lc, kth, 0.0), axis=0, keepdims=True)
    kth_lane_ref[:, pl.ds(i * r, r)] = kl

    @pl.when(i == 0)
    def _init():
        colsel_ref[...] = jnp.zeros_like(colsel_ref)

    colsel_ref[...] = jnp.maximum(colsel_ref[...], hits)

    @pl.when(i == n - 1)
    def _sigma():
        x = kth_lane_ref[...]

        def body(_, lohi):
            lo, hi = lohi
            mid = lo + ((hi - lo) >> 1)
            t = jax.lax.bitcast_convert_type(
                jnp.full((1, B), mid, jnp.int32), jnp.float32)
            cnt = jnp.sum((x <= t).astype(jnp.float32))
            return (jnp.where(cnt >= float(MED_RANK), lo, mid + 1),
                    jnp.where(cnt >= float(MED_RANK), mid, hi))

        lo, _ = jax.lax.fori_loop(
            0, 31, body, (jnp.int32(0), jnp.int32(0x7F800000)))
        med_d2 = jax.lax.bitcast_convert_type(lo, jnp.float32)
        sigma_ref[...] = jnp.broadcast_to(jnp.sqrt(jnp.sqrt(med_d2)), (1, 1))


def _aff_kernel(sigma_ref, colsel_ref, dist_ref, out_ref):
    i = pl.program_id(0)
    sigma = sigma_ref[0, 0]
    neg_inv = -1.0 / (2.0 * sigma * sigma)

    dist = dist_ref[...]
    r = dist.shape[0]
    e = jnp.exp(dist * neg_inv)

    cs_lane = colsel_ref[...]
    cs_chunk = colsel_ref[:, pl.ds(i * r, r)]
    lr = jax.lax.broadcasted_iota(jnp.int32, (r, r), 0)
    lc = jax.lax.broadcasted_iota(jnp.int32, (r, r), 1)
    cs_row = jnp.sum(jnp.where(lr == lc, cs_chunk, 0.0), axis=1,
                     keepdims=True)
    out_ref[...] = e * (cs_row + cs_lane)


def kernel(inp):
    n1 = B // ROWS1
    dist, kth_lane, colsel, sigma = pl.pallas_call(
        _knn_kernel,
        grid=(n1,),
        in_specs=[
            pl.BlockSpec((B, D), lambda i: (0, 0)),
            pl.BlockSpec((ROWS1, D), lambda i: (i, 0)),
        ],
        out_specs=[
            pl.BlockSpec((ROWS1, B), lambda i: (i, 0)),
            pl.BlockSpec((1, B), lambda i: (0, 0)),
            pl.BlockSpec((1, B), lambda i: (0, 0)),
            pl.BlockSpec((1, 1), lambda i: (0, 0)),
        ],
        out_shape=[
            jax.ShapeDtypeStruct((B, B), jnp.float32),
            jax.ShapeDtypeStruct((1, B), jnp.float32),
            jax.ShapeDtypeStruct((1, B), jnp.float32),
            jax.ShapeDtypeStruct((1, 1), jnp.float32),
        ],
        compiler_params=pltpu.CompilerParams(
            dimension_semantics=("arbitrary",)),
    )(inp, inp)

    n2 = B // ROWS2
    sym = pl.pallas_call(
        _aff_kernel,
        grid=(n2,),
        in_specs=[
            pl.BlockSpec((1, 1), lambda i: (0, 0)),
            pl.BlockSpec((1, B), lambda i: (0, 0)),
            pl.BlockSpec((ROWS2, B), lambda i: (i, 0)),
        ],
        out_specs=pl.BlockSpec((ROWS2, B), lambda i: (i, 0)),
        out_shape=jax.ShapeDtypeStruct((B, B), jnp.float32),
        compiler_params=pltpu.CompilerParams(
            dimension_semantics=("arbitrary",)),
    )(sigma, colsel, dist)
    return sym


# speedup vs baseline: 5.6302x; 1.0001x over previous
import jax
import jax.numpy as jnp
from jax.experimental import pallas as pl
from jax.experimental.pallas import tpu as pltpu

B = 4096
D = 64
NK = 10
KTH = 7
BIG = 1000000000.0
ROWS1 = 512
ROWS2 = 512
MED_RANK = B // 2 + (B % 2)


def _knn_kernel(inp_ref, rows_ref, dist_ref, kth_lane_ref, colsel_ref,
                sigma_ref):
    i = pl.program_id(0)
    n = pl.num_programs(0)
    allx = inp_ref[...]
    rows = rows_ref[...]
    r = rows.shape[0]

    gram = jax.lax.dot_general(rows, allx, (((1,), (1,)), ((), ())),
                               preferred_element_type=jnp.float32)
    rsq = jnp.sum(rows * rows, axis=1, keepdims=True)
    ones = jnp.ones((1, D), jnp.float32)
    csq = jax.lax.dot_general(ones, allx * allx, (((1,), (1,)), ((), ())),
                              preferred_element_type=jnp.float32,
                              precision=jax.lax.Precision.HIGHEST)
    d2raw = jnp.maximum(rsq + csq - 2.0 * gram, 0.0)

    row_g = i * r + jax.lax.broadcasted_iota(jnp.int32, (r, B), 0)
    col_i = jax.lax.broadcasted_iota(jnp.int32, (r, B), 1)
    diag = row_g == col_i
    d2 = jnp.where(diag, BIG, d2raw)
    dist_ref[...] = jnp.where(diag, BIG, jnp.sqrt(d2raw))

    m = jnp.min(d2, axis=1, keepdims=True)
    kth = m
    for k in range(1, NK):
        m = jnp.min(jnp.where(d2 <= m, BIG, d2), axis=1, keepdims=True)
        if k == KTH:
            kth = m
    t10 = m

    anyhit = jnp.any(d2 <= t10, axis=0, keepdims=True)
    hits = jnp.where(anyhit, 0.5, 0.0)

    lr = jax.lax.broadcasted_iota(jnp.int32, (r, r), 0)
    lc = jax.lax.broadcasted_iota(jnp.int32, (r, r), 1)
    kl = jnp.sum(jnp.where(lr == lc, kth, 0.0), axis=0, keepdims=True)
    kth_lane_ref[:, pl.ds(i * r, r)] = kl

    @pl.when(i == 0)
    def _init():
        colsel_ref[...] = jnp.zeros_like(colsel_ref)

    colsel_ref[...] = jnp.maximum(colsel_ref[...], hits)

    @pl.when(i == n - 1)
    def _sigma():
        x = kth_lane_ref[...]

        def body(_, lohi):
            lo, hi = lohi
            mid = lo + ((hi - lo) >> 1)
            t = jax.lax.bitcast_convert_type(
                jnp.full((1, B), mid, jnp.int32), jnp.float32)
            cnt = jnp.sum((x <= t).astype(jnp.float32))
            return (jnp.where(cnt >= float(MED_RANK), lo, mid + 1),
                    jnp.where(cnt >= float(MED_RANK), mid, hi))

        lo, _ = jax.lax.fori_loop(
            0, 31, body, (jnp.int32(0), jnp.int32(0x7F800000)))
        med_d2 = jax.lax.bitcast_convert_type(lo, jnp.float32)
        sigma_ref[...] = jnp.broadcast_to(jnp.sqrt(jnp.sqrt(med_d2)), (1, 1))


def _aff_kernel(sigma_ref, colsel_ref, dist_ref, out_ref):
    i = pl.program_id(0)
    sigma = sigma_ref[0, 0]
    neg_inv = -1.0 / (2.0 * sigma * sigma)

    dist = dist_ref[...]
    r = dist.shape[0]
    e = jnp.exp(dist * neg_inv)

    cs_lane = colsel_ref[...]
    cs_chunk = colsel_ref[:, pl.ds(i * r, r)]
    lr = jax.lax.broadcasted_iota(jnp.int32, (r, r), 0)
    lc = jax.lax.broadcasted_iota(jnp.int32, (r, r), 1)
    cs_row = jnp.sum(jnp.where(lr == lc, cs_chunk, 0.0), axis=1,
                     keepdims=True)
    out_ref[...] = e * (cs_row + cs_lane)


def kernel(inp):
    n1 = B // ROWS1
    dist, kth_lane, colsel, sigma = pl.pallas_call(
        _knn_kernel,
        grid=(n1,),
        in_specs=[
            pl.BlockSpec((B, D), lambda i: (0, 0)),
            pl.BlockSpec((ROWS1, D), lambda i: (i, 0)),
        ],
        out_specs=[
            pl.BlockSpec((ROWS1, B), lambda i: (i, 0)),
            pl.BlockSpec((1, B), lambda i: (0, 0)),
            pl.BlockSpec((1, B), lambda i: (0, 0)),
            pl.BlockSpec((1, 1), lambda i: (0, 0)),
        ],
        out_shape=[
            jax.ShapeDtypeStruct((B, B), jnp.float32),
            jax.ShapeDtypeStruct((1, B), jnp.float32),
            jax.ShapeDtypeStruct((1, B), jnp.float32),
            jax.ShapeDtypeStruct((1, 1), jnp.float32),
        ],
        compiler_params=pltpu.CompilerParams(
            dimension_semantics=("arbitrary",)),
    )(inp, inp)

    n2 = B // ROWS2
    sym = pl.pallas_call(
        _aff_kernel,
        grid=(n2,),
        in_specs=[
            pl.BlockSpec((1, 1), lambda i: (0, 0)),
            pl.BlockSpec((1, B), lambda i: (0, 0)),
            pl.BlockSpec((ROWS2, B), lambda i: (i, 0)),
        ],
        out_specs=pl.BlockSpec((ROWS2, B), lambda i: (i, 0)),
        out_shape=jax.ShapeDtypeStruct((B, B), jnp.float32),
        compiler_params=pltpu.CompilerParams(
            dimension_semantics=("parallel",)),
    )(sigma, colsel, dist)
    return sym


# speedup vs baseline: 6.1409x; 1.0907x over previous
import jax
import jax.numpy as jnp
from jax.experimental import pallas as pl
from jax.experimental.pallas import tpu as pltpu

B = 4096
D = 64
NK = 10
KTH = 7
BIG = 1000000000.0
ROWS1 = 512
ROWS2 = 512
MED_RANK = B // 2 + (B % 2)


def _knn_kernel(inp_ref, rows_ref, dist_ref, kth_lane_ref, colsel_ref,
                sigma_ref):
    i = pl.program_id(0)
    n = pl.num_programs(0)
    allx = inp_ref[...]
    rows = rows_ref[...]
    r = rows.shape[0]

    gram = jax.lax.dot_general(rows, allx, (((1,), (1,)), ((), ())),
                               preferred_element_type=jnp.float32)
    rsq = jnp.sum(rows * rows, axis=1, keepdims=True)
    ones = jnp.ones((1, D), jnp.float32)
    csq = jax.lax.dot_general(ones, allx * allx, (((1,), (1,)), ((), ())),
                              preferred_element_type=jnp.float32,
                              precision=jax.lax.Precision.HIGHEST)
    d2raw = jnp.maximum(rsq + csq - 2.0 * gram, 0.0)

    row_g = i * r + jax.lax.broadcasted_iota(jnp.int32, (r, B), 0)
    col_i = jax.lax.broadcasted_iota(jnp.int32, (r, B), 1)
    d2 = jnp.where(row_g == col_i, BIG, d2raw)
    dist_ref[...] = d2

    m = jnp.min(d2, axis=1, keepdims=True)
    kth = m
    for k in range(1, NK):
        m = jnp.min(jnp.where(d2 <= m, BIG, d2), axis=1, keepdims=True)
        if k == KTH:
            kth = m
    t10 = m

    anyhit = jnp.any(d2 <= t10, axis=0, keepdims=True)
    hits = jnp.where(anyhit, 0.5, 0.0)

    lr = jax.lax.broadcasted_iota(jnp.int32, (r, r), 0)
    lc = jax.lax.broadcasted_iota(jnp.int32, (r, r), 1)
    kl = jnp.sum(jnp.where(lr == lc, kth, 0.0), axis=0, keepdims=True)
    kth_lane_ref[:, pl.ds(i * r, r)] = kl

    @pl.when(i == 0)
    def _init():
        colsel_ref[...] = jnp.zeros_like(colsel_ref)

    colsel_ref[...] = jnp.maximum(colsel_ref[...], hits)

    @pl.when(i == n - 1)
    def _sigma():
        x = kth_lane_ref[...]

        def body(_, lohi):
            lo, hi = lohi
            mid = lo + ((hi - lo) >> 1)
            t = jax.lax.bitcast_convert_type(
                jnp.full((1, B), mid, jnp.int32), jnp.float32)
            cnt = jnp.sum((x <= t).astype(jnp.float32))
            return (jnp.where(cnt >= float(MED_RANK), lo, mid + 1),
                    jnp.where(cnt >= float(MED_RANK), mid, hi))

        lo, _ = jax.lax.fori_loop(
            0, 31, body, (jnp.int32(0), jnp.int32(0x7F800000)))
        med_d2 = jax.lax.bitcast_convert_type(lo, jnp.float32)
        sigma_ref[...] = jnp.broadcast_to(jnp.sqrt(jnp.sqrt(med_d2)), (1, 1))


def _aff_kernel(sigma_ref, colsel_ref, dist_ref, out_ref):
    i = pl.program_id(0)
    sigma = sigma_ref[0, 0]
    neg_inv = -1.0 / (2.0 * sigma * sigma)

    d2 = dist_ref[...]
    r = d2.shape[0]
    e = jnp.exp(jnp.sqrt(d2) * neg_inv)

    cs_lane = colsel_ref[...]
    cs_chunk = colsel_ref[:, pl.ds(i * r, r)]
    lr = jax.lax.broadcasted_iota(jnp.int32, (r, r), 0)
    lc = jax.lax.broadcasted_iota(jnp.int32, (r, r), 1)
    cs_row = jnp.sum(jnp.where(lr == lc, cs_chunk, 0.0), axis=1,
                     keepdims=True)
    out_ref[...] = e * (cs_row + cs_lane)


def kernel(inp):
    n1 = B // ROWS1
    dist, kth_lane, colsel, sigma = pl.pallas_call(
        _knn_kernel,
        grid=(n1,),
        in_specs=[
            pl.BlockSpec((B, D), lambda i: (0, 0)),
            pl.BlockSpec((ROWS1, D), lambda i: (i, 0)),
        ],
        out_specs=[
            pl.BlockSpec((ROWS1, B), lambda i: (i, 0)),
            pl.BlockSpec((1, B), lambda i: (0, 0)),
            pl.BlockSpec((1, B), lambda i: (0, 0)),
            pl.BlockSpec((1, 1), lambda i: (0, 0)),
        ],
        out_shape=[
            jax.ShapeDtypeStruct((B, B), jnp.float32),
            jax.ShapeDtypeStruct((1, B), jnp.float32),
            jax.ShapeDtypeStruct((1, B), jnp.float32),
            jax.ShapeDtypeStruct((1, 1), jnp.float32),
        ],
        compiler_params=pltpu.CompilerParams(
            dimension_semantics=("arbitrary",)),
    )(inp, inp)

    n2 = B // ROWS2
    sym = pl.pallas_call(
        _aff_kernel,
        grid=(n2,),
        in_specs=[
            pl.BlockSpec((1, 1), lambda i: (0, 0)),
            pl.BlockSpec((1, B), lambda i: (0, 0)),
            pl.BlockSpec((ROWS2, B), lambda i: (i, 0)),
        ],
        out_specs=pl.BlockSpec((ROWS2, B), lambda i: (i, 0)),
        out_shape=jax.ShapeDtypeStruct((B, B), jnp.float32),
        compiler_params=pltpu.CompilerParams(
            dimension_semantics=("parallel",)),
    )(sigma, colsel, dist)
    return sym


# speedup vs baseline: 6.8949x; 1.1228x over previous
import jax
import jax.numpy as jnp
from jax.experimental import pallas as pl
from jax.experimental.pallas import tpu as pltpu

B = 4096
D = 64
NK = 10
KTH = 7
BIG = 1000000000.0
ROWS1 = 512
ROWS2 = 512
MED_RANK = B // 2 + (B % 2)


def _knn_kernel(inp_ref, rows_ref, dist_ref, kth_lane_ref, colsel_ref,
                sigma_ref, kth_scr, t10_scr):
    i = pl.program_id(0)
    n = pl.num_programs(0)
    allx = inp_ref[...]
    rows = rows_ref[...]
    r = rows.shape[0]

    gram = jax.lax.dot_general(rows, allx, (((1,), (1,)), ((), ())),
                               preferred_element_type=jnp.float32)
    rsq = jnp.sum(rows * rows, axis=1, keepdims=True)
    ones = jnp.ones((1, D), jnp.float32)
    csq = jax.lax.dot_general(ones, allx * allx, (((1,), (1,)), ((), ())),
                              preferred_element_type=jnp.float32,
                              precision=jax.lax.Precision.HIGHEST)
    d2raw = jnp.maximum(rsq + csq - 2.0 * gram, 0.0)

    row_g = i * r + jax.lax.broadcasted_iota(jnp.int32, (r, B), 0)
    col_i = jax.lax.broadcasted_iota(jnp.int32, (r, B), 1)
    d2 = jnp.where(row_g == col_i, BIG, d2raw)
    dist_ref[...] = d2

    c = B // 8
    a = [d2[:, j * c:(j + 1) * c] for j in range(8)]
    s = [jnp.minimum(a[2 * j], a[2 * j + 1]) for j in range(4)]
    l = [jnp.maximum(a[2 * j], a[2 * j + 1]) for j in range(4)]

    def _bot3(s1, l1, s2, l2):
        m_ = jnp.minimum(s1, s2)
        mm = jnp.maximum(s1, s2)
        n_ = jnp.minimum(l1, l2)
        return m_, jnp.minimum(mm, n_), jnp.maximum(mm, n_)

    x1, x2, x3 = _bot3(s[0], l[0], s[1], l[1])
    y1, y2, y3 = _bot3(s[2], l[2], s[3], l[3])
    z1 = jnp.minimum(x1, y1)
    za = jnp.maximum(x1, y1)
    zb = jnp.minimum(x2, y2)
    z2 = jnp.minimum(za, zb)
    z3 = jnp.minimum(jnp.maximum(za, zb), jnp.minimum(x3, y3))
    cand = jnp.concatenate([z1, z2, z3], axis=1)

    m = jnp.min(cand, axis=1, keepdims=True)
    kth = m
    for k in range(1, NK):
        m = jnp.min(jnp.where(cand <= m, BIG, cand), axis=1, keepdims=True)
        if k == KTH:
            kth = m
    t10 = m

    below = (d2 <= t10).astype(jnp.float32)
    cnt = below[:, 0:c]
    for j in range(1, 8):
        cnt = cnt + below[:, j * c:(j + 1) * c]
    bad = jnp.max(cnt) >= 4.0

    kth_scr[...] = kth
    t10_scr[...] = t10

    @pl.when(bad)
    def _fallback():
        mm = jnp.min(d2, axis=1, keepdims=True)
        kk = mm
        for k in range(1, NK):
            mm = jnp.min(jnp.where(d2 <= mm, BIG, d2), axis=1,
                         keepdims=True)
            if k == KTH:
                kk = mm
        kth_scr[...] = kk
        t10_scr[...] = mm

    kth = kth_scr[...]
    t10 = t10_scr[...]

    anyhit = jnp.any(d2 <= t10, axis=0, keepdims=True)
    hits = jnp.where(anyhit, 0.5, 0.0)

    lr = jax.lax.broadcasted_iota(jnp.int32, (r, r), 0)
    lc = jax.lax.broadcasted_iota(jnp.int32, (r, r), 1)
    kl = jnp.sum(jnp.where(lr == lc, kth, 0.0), axis=0, keepdims=True)
    kth_lane_ref[:, pl.ds(i * r, r)] = kl

    @pl.when(i == 0)
    def _init():
        colsel_ref[...] = jnp.zeros_like(colsel_ref)

    colsel_ref[...] = jnp.maximum(colsel_ref[...], hits)

    @pl.when(i == n - 1)
    def _sigma():
        x = kth_lane_ref[...]

        def body(_, lohi):
            lo, hi = lohi
            mid = lo + ((hi - lo) >> 1)
            t = jax.lax.bitcast_convert_type(
                jnp.full((1, B), mid, jnp.int32), jnp.float32)
            cnt = jnp.sum((x <= t).astype(jnp.float32))
            return (jnp.where(cnt >= float(MED_RANK), lo, mid + 1),
                    jnp.where(cnt >= float(MED_RANK), mid, hi))

        lo, _ = jax.lax.fori_loop(
            0, 31, body, (jnp.int32(0), jnp.int32(0x7F800000)))
        med_d2 = jax.lax.bitcast_convert_type(lo, jnp.float32)
        sigma_ref[...] = jnp.broadcast_to(jnp.sqrt(jnp.sqrt(med_d2)), (1, 1))


def _aff_kernel(sigma_ref, colsel_ref, dist_ref, out_ref):
    i = pl.program_id(0)
    sigma = sigma_ref[0, 0]
    neg_inv = -1.0 / (2.0 * sigma * sigma)

    d2 = dist_ref[...]
    r = d2.shape[0]
    e = jnp.exp(jnp.sqrt(d2) * neg_inv)

    cs_lane = colsel_ref[...]
    cs_chunk = colsel_ref[:, pl.ds(i * r, r)]
    lr = jax.lax.broadcasted_iota(jnp.int32, (r, r), 0)
    lc = jax.lax.broadcasted_iota(jnp.int32, (r, r), 1)
    cs_row = jnp.sum(jnp.where(lr == lc, cs_chunk, 0.0), axis=1,
                     keepdims=True)
    out_ref[...] = e * (cs_row + cs_lane)


def kernel(inp):
    n1 = B // ROWS1
    dist, kth_lane, colsel, sigma = pl.pallas_call(
        _knn_kernel,
        grid=(n1,),
        in_specs=[
            pl.BlockSpec((B, D), lambda i: (0, 0)),
            pl.BlockSpec((ROWS1, D), lambda i: (i, 0)),
        ],
        out_specs=[
            pl.BlockSpec((ROWS1, B), lambda i: (i, 0)),
            pl.BlockSpec((1, B), lambda i: (0, 0)),
            pl.BlockSpec((1, B), lambda i: (0, 0)),
            pl.BlockSpec((1, 1), lambda i: (0, 0)),
        ],
        out_shape=[
            jax.ShapeDtypeStruct((B, B), jnp.float32),
            jax.ShapeDtypeStruct((1, B), jnp.float32),
            jax.ShapeDtypeStruct((1, B), jnp.float32),
            jax.ShapeDtypeStruct((1, 1), jnp.float32),
        ],
        scratch_shapes=[
            pltpu.VMEM((ROWS1, 1), jnp.float32),
            pltpu.VMEM((ROWS1, 1), jnp.float32),
        ],
        compiler_params=pltpu.CompilerParams(
            dimension_semantics=("arbitrary",)),
    )(inp, inp)

    n2 = B // ROWS2
    sym = pl.pallas_call(
        _aff_kernel,
        grid=(n2,),
        in_specs=[
            pl.BlockSpec((1, 1), lambda i: (0, 0)),
            pl.BlockSpec((1, B), lambda i: (0, 0)),
            pl.BlockSpec((ROWS2, B), lambda i: (i, 0)),
        ],
        out_specs=pl.BlockSpec((ROWS2, B), lambda i: (i, 0)),
        out_shape=jax.ShapeDtypeStruct((B, B), jnp.float32),
        compiler_params=pltpu.CompilerParams(
            dimension_semantics=("parallel",)),
    )(sigma, colsel, dist)
    return sym


# speedup vs baseline: 7.0440x; 1.0216x over previous
import jax
import jax.numpy as jnp
from jax.experimental import pallas as pl
from jax.experimental.pallas import tpu as pltpu

B = 4096
D = 64
NK = 10
KTH = 7
BIG = 1000000000.0
ROWS1 = 512
ROWS2 = 512
MED_RANK = B // 2 + (B % 2)


def _knn_kernel(inp_ref, rows_ref, dist_ref, kth_lane_ref, colsel_ref,
                sigma_ref, kth_scr, hits_scr):
    i = pl.program_id(0)
    n = pl.num_programs(0)
    allx = inp_ref[...]
    rows = rows_ref[...]
    r = rows.shape[0]

    gram = jax.lax.dot_general(rows, allx, (((1,), (1,)), ((), ())),
                               preferred_element_type=jnp.float32)
    rsq = jnp.sum(rows * rows, axis=1, keepdims=True)
    ones = jnp.ones((1, D), jnp.float32)
    csq = jax.lax.dot_general(ones, allx * allx, (((1,), (1,)), ((), ())),
                              preferred_element_type=jnp.float32,
                              precision=jax.lax.Precision.HIGHEST)
    d2raw = jnp.maximum(rsq + csq - 2.0 * gram, 0.0)

    row_g = i * r + jax.lax.broadcasted_iota(jnp.int32, (r, B), 0)
    col_i = jax.lax.broadcasted_iota(jnp.int32, (r, B), 1)
    d2 = jnp.where(row_g == col_i, BIG, d2raw)
    dist_ref[...] = d2

    c = B // 8
    a = [d2[:, j * c:(j + 1) * c] for j in range(8)]
    s = [jnp.minimum(a[2 * j], a[2 * j + 1]) for j in range(4)]
    l = [jnp.maximum(a[2 * j], a[2 * j + 1]) for j in range(4)]

    def _sort4(s1, l1, s2, l2):
        m_ = jnp.minimum(s1, s2)
        mm = jnp.maximum(s1, s2)
        n_ = jnp.minimum(l1, l2)
        return (m_, jnp.minimum(mm, n_), jnp.maximum(mm, n_),
                jnp.maximum(l1, l2))

    x1, x2, x3, x4 = _sort4(s[0], l[0], s[1], l[1])
    y1, y2, y3, y4 = _sort4(s[2], l[2], s[3], l[3])
    z1 = jnp.minimum(x1, y1)
    za = jnp.maximum(x1, y1)
    zb = jnp.minimum(x2, y2)
    z2 = jnp.minimum(za, zb)
    z3 = jnp.minimum(jnp.maximum(za, zb), jnp.minimum(x3, y3))
    z4 = jnp.minimum(
        jnp.minimum(jnp.minimum(x4, y4),
                    jnp.maximum(x3, y1)),
        jnp.minimum(jnp.maximum(x2, y2), jnp.maximum(x1, y3)))
    cand = jnp.concatenate([z1, z2, z3], axis=1)

    m = jnp.min(cand, axis=1, keepdims=True)
    kth = m
    for k in range(1, NK):
        m = jnp.min(jnp.where(cand <= m, BIG, cand), axis=1, keepdims=True)
        if k == KTH:
            kth = m
    t10 = m

    bad = jnp.any(z4 <= t10)

    kth_scr[...] = kth
    anyhit = jnp.any(d2 <= t10, axis=0, keepdims=True)
    hits_scr[...] = jnp.where(anyhit, 0.5, 0.0)

    @pl.when(bad)
    def _fallback():
        mm = jnp.min(d2, axis=1, keepdims=True)
        kk = mm
        for k in range(1, NK):
            mm = jnp.min(jnp.where(d2 <= mm, BIG, d2), axis=1,
                         keepdims=True)
            if k == KTH:
                kk = mm
        kth_scr[...] = kk
        ah = jnp.any(d2 <= mm, axis=0, keepdims=True)
        hits_scr[...] = jnp.where(ah, 0.5, 0.0)

    kth = kth_scr[...]
    hits = hits_scr[...]

    lr = jax.lax.broadcasted_iota(jnp.int32, (r, r), 0)
    lc = jax.lax.broadcasted_iota(jnp.int32, (r, r), 1)
    kl = jnp.sum(jnp.where(lr == lc, kth, 0.0), axis=0, keepdims=True)
    kth_lane_ref[:, pl.ds(i * r, r)] = kl

    @pl.when(i == 0)
    def _init():
        colsel_ref[...] = jnp.zeros_like(colsel_ref)

    colsel_ref[...] = jnp.maximum(colsel_ref[...], hits)

    @pl.when(i == n - 1)
    def _sigma():
        x = kth_lane_ref[...]

        def body(_, lohi):
            lo, hi = lohi
            mid = lo + ((hi - lo) >> 1)
            t = jax.lax.bitcast_convert_type(
                jnp.full((1, B), mid, jnp.int32), jnp.float32)
            cnt = jnp.sum((x <= t).astype(jnp.float32))
            return (jnp.where(cnt >= float(MED_RANK), lo, mid + 1),
                    jnp.where(cnt >= float(MED_RANK), mid, hi))

        lo, _ = jax.lax.fori_loop(
            0, 31, body, (jnp.int32(0), jnp.int32(0x7F800000)))
        med_d2 = jax.lax.bitcast_convert_type(lo, jnp.float32)
        sigma_ref[...] = jnp.broadcast_to(jnp.sqrt(jnp.sqrt(med_d2)), (1, 1))


def _aff_kernel(sigma_ref, colsel_ref, dist_ref, out_ref):
    i = pl.program_id(0)
    sigma = sigma_ref[0, 0]
    neg_inv = -1.0 / (2.0 * sigma * sigma)

    d2 = dist_ref[...]
    r = d2.shape[0]
    e = jnp.exp(jnp.sqrt(d2) * neg_inv)

    cs_lane = colsel_ref[...]
    cs_chunk = colsel_ref[:, pl.ds(i * r, r)]
    lr = jax.lax.broadcasted_iota(jnp.int32, (r, r), 0)
    lc = jax.lax.broadcasted_iota(jnp.int32, (r, r), 1)
    cs_row = jnp.sum(jnp.where(lr == lc, cs_chunk, 0.0), axis=1,
                     keepdims=True)
    out_ref[...] = e * (cs_row + cs_lane)


def kernel(inp):
    n1 = B // ROWS1
    dist, kth_lane, colsel, sigma = pl.pallas_call(
        _knn_kernel,
        grid=(n1,),
        in_specs=[
            pl.BlockSpec((B, D), lambda i: (0, 0)),
            pl.BlockSpec((ROWS1, D), lambda i: (i, 0)),
        ],
        out_specs=[
            pl.BlockSpec((ROWS1, B), lambda i: (i, 0)),
            pl.BlockSpec((1, B), lambda i: (0, 0)),
            pl.BlockSpec((1, B), lambda i: (0, 0)),
            pl.BlockSpec((1, 1), lambda i: (0, 0)),
        ],
        out_shape=[
            jax.ShapeDtypeStruct((B, B), jnp.float32),
            jax.ShapeDtypeStruct((1, B), jnp.float32),
            jax.ShapeDtypeStruct((1, B), jnp.float32),
            jax.ShapeDtypeStruct((1, 1), jnp.float32),
        ],
        scratch_shapes=[
            pltpu.VMEM((ROWS1, 1), jnp.float32),
            pltpu.VMEM((1, B), jnp.float32),
        ],
        compiler_params=pltpu.CompilerParams(
            dimension_semantics=("arbitrary",)),
    )(inp, inp)

    n2 = B // ROWS2
    sym = pl.pallas_call(
        _aff_kernel,
        grid=(n2,),
        in_specs=[
            pl.BlockSpec((1, 1), lambda i: (0, 0)),
            pl.BlockSpec((1, B), lambda i: (0, 0)),
            pl.BlockSpec((ROWS2, B), lambda i: (i, 0)),
        ],
        out_specs=pl.BlockSpec((ROWS2, B), lambda i: (i, 0)),
        out_shape=jax.ShapeDtypeStruct((B, B), jnp.float32),
        compiler_params=pltpu.CompilerParams(
            dimension_semantics=("parallel",)),
    )(sigma, colsel, dist)
    return sym


# speedup vs baseline: 7.4212x; 1.0536x over previous
import jax
import jax.numpy as jnp
from jax.experimental import pallas as pl
from jax.experimental.pallas import tpu as pltpu

B = 4096
D = 64
NK = 10
KTH = 7
BIG = 1000000000.0
ROWS1 = 512
ROWS2 = 512
MED_RANK = B // 2 + (B % 2)


def _knn_kernel(inp_ref, rows_ref, dist_ref, kth_lane_ref, colsel_ref,
                sigma_ref, kth_scr, hits_scr):
    i = pl.program_id(0)
    n = pl.num_programs(0)
    allx = inp_ref[...]
    rows = rows_ref[...]
    r = rows.shape[0]

    gram = jax.lax.dot_general(rows, allx, (((1,), (1,)), ((), ())),
                               preferred_element_type=jnp.float32)
    rsq = jnp.sum(rows * rows, axis=1, keepdims=True)
    ones = jnp.ones((1, D), jnp.float32)
    csq = jax.lax.dot_general(ones, allx * allx, (((1,), (1,)), ((), ())),
                              preferred_element_type=jnp.float32,
                              precision=jax.lax.Precision.HIGHEST)
    d2raw = jnp.maximum(rsq + csq - 2.0 * gram, 0.0)

    row_g = i * r + jax.lax.broadcasted_iota(jnp.int32, (r, B), 0)
    col_i = jax.lax.broadcasted_iota(jnp.int32, (r, B), 1)
    d2 = jnp.where(row_g == col_i, BIG, d2raw)
    dist_ref[...] = d2.astype(jnp.bfloat16)

    c = B // 8
    a = [d2[:, j * c:(j + 1) * c] for j in range(8)]
    s = [jnp.minimum(a[2 * j], a[2 * j + 1]) for j in range(4)]
    l = [jnp.maximum(a[2 * j], a[2 * j + 1]) for j in range(4)]

    def _sort4(s1, l1, s2, l2):
        m_ = jnp.minimum(s1, s2)
        mm = jnp.maximum(s1, s2)
        n_ = jnp.minimum(l1, l2)
        return (m_, jnp.minimum(mm, n_), jnp.maximum(mm, n_),
                jnp.maximum(l1, l2))

    x1, x2, x3, x4 = _sort4(s[0], l[0], s[1], l[1])
    y1, y2, y3, y4 = _sort4(s[2], l[2], s[3], l[3])
    z1 = jnp.minimum(x1, y1)
    za = jnp.maximum(x1, y1)
    zb = jnp.minimum(x2, y2)
    z2 = jnp.minimum(za, zb)
    z3 = jnp.minimum(jnp.maximum(za, zb), jnp.minimum(x3, y3))
    z4 = jnp.minimum(
        jnp.minimum(jnp.minimum(x4, y4),
                    jnp.maximum(x3, y1)),
        jnp.minimum(jnp.maximum(x2, y2), jnp.maximum(x1, y3)))
    cand = jnp.concatenate([z1, z2, z3], axis=1)

    m = jnp.min(cand, axis=1, keepdims=True)
    kth = m
    for k in range(1, NK):
        m = jnp.min(jnp.where(cand <= m, BIG, cand), axis=1, keepdims=True)
        if k == KTH:
            kth = m
    t10 = m

    bad = jnp.any(z4 <= t10)

    kth_scr[...] = kth
    anyhit = jnp.any(d2 <= t10, axis=0, keepdims=True)
    hits_scr[...] = jnp.where(anyhit, 0.5, 0.0)

    @pl.when(bad)
    def _fallback():
        mm = jnp.min(d2, axis=1, keepdims=True)
        kk = mm
        for k in range(1, NK):
            mm = jnp.min(jnp.where(d2 <= mm, BIG, d2), axis=1,
                         keepdims=True)
            if k == KTH:
                kk = mm
        kth_scr[...] = kk
        ah = jnp.any(d2 <= mm, axis=0, keepdims=True)
        hits_scr[...] = jnp.where(ah, 0.5, 0.0)

    kth = kth_scr[...]
    hits = hits_scr[...]

    lr = jax.lax.broadcasted_iota(jnp.int32, (r, r), 0)
    lc = jax.lax.broadcasted_iota(jnp.int32, (r, r), 1)
    kl = jnp.sum(jnp.where(lr == lc, kth, 0.0), axis=0, keepdims=True)
    kth_lane_ref[:, pl.ds(i * r, r)] = kl

    @pl.when(i == 0)
    def _init():
        colsel_ref[...] = jnp.zeros_like(colsel_ref)

    colsel_ref[...] = jnp.maximum(colsel_ref[...], hits)

    @pl.when(i == n - 1)
    def _sigma():
        x = kth_lane_ref[...]

        def body(_, lohi):
            lo, hi = lohi
            mid = lo + ((hi - lo) >> 1)
            t = jax.lax.bitcast_convert_type(
                jnp.full((1, B), mid, jnp.int32), jnp.float32)
            cnt = jnp.sum((x <= t).astype(jnp.float32))
            return (jnp.where(cnt >= float(MED_RANK), lo, mid + 1),
                    jnp.where(cnt >= float(MED_RANK), mid, hi))

        lo, _ = jax.lax.fori_loop(
            0, 31, body, (jnp.int32(0), jnp.int32(0x7F800000)))
        med_d2 = jax.lax.bitcast_convert_type(lo, jnp.float32)
        sigma_ref[...] = jnp.broadcast_to(jnp.sqrt(jnp.sqrt(med_d2)), (1, 1))


def _aff_kernel(sigma_ref, colsel_ref, dist_ref, out_ref):
    i = pl.program_id(0)
    sigma = sigma_ref[0, 0]
    neg_inv = -1.0 / (2.0 * sigma * sigma)

    d2 = dist_ref[...].astype(jnp.float32)
    r = d2.shape[0]
    e = jnp.exp(jnp.sqrt(d2) * neg_inv)

    cs_lane = colsel_ref[...]
    cs_chunk = colsel_ref[:, pl.ds(i * r, r)]
    lr = jax.lax.broadcasted_iota(jnp.int32, (r, r), 0)
    lc = jax.lax.broadcasted_iota(jnp.int32, (r, r), 1)
    cs_row = jnp.sum(jnp.where(lr == lc, cs_chunk, 0.0), axis=1,
                     keepdims=True)
    out_ref[...] = e * (cs_row + cs_lane)


def kernel(inp):
    n1 = B // ROWS1
    dist, kth_lane, colsel, sigma = pl.pallas_call(
        _knn_kernel,
        grid=(n1,),
        in_specs=[
            pl.BlockSpec((B, D), lambda i: (0, 0)),
            pl.BlockSpec((ROWS1, D), lambda i: (i, 0)),
        ],
        out_specs=[
            pl.BlockSpec((ROWS1, B), lambda i: (i, 0)),
            pl.BlockSpec((1, B), lambda i: (0, 0)),
            pl.BlockSpec((1, B), lambda i: (0, 0)),
            pl.BlockSpec((1, 1), lambda i: (0, 0)),
        ],
        out_shape=[
            jax.ShapeDtypeStruct((B, B), jnp.bfloat16),
            jax.ShapeDtypeStruct((1, B), jnp.float32),
            jax.ShapeDtypeStruct((1, B), jnp.float32),
            jax.ShapeDtypeStruct((1, 1), jnp.float32),
        ],
        scratch_shapes=[
            pltpu.VMEM((ROWS1, 1), jnp.float32),
            pltpu.VMEM((1, B), jnp.float32),
        ],
        compiler_params=pltpu.CompilerParams(
            dimension_semantics=("arbitrary",)),
    )(inp, inp)

    n2 = B // ROWS2
    sym = pl.pallas_call(
        _aff_kernel,
        grid=(n2,),
        in_specs=[
            pl.BlockSpec((1, 1), lambda i: (0, 0)),
            pl.BlockSpec((1, B), lambda i: (0, 0)),
            pl.BlockSpec((ROWS2, B), lambda i: (i, 0)),
        ],
        out_specs=pl.BlockSpec((ROWS2, B), lambda i: (i, 0)),
        out_shape=jax.ShapeDtypeStruct((B, B), jnp.float32),
        compiler_params=pltpu.CompilerParams(
            dimension_semantics=("parallel",)),
    )(sigma, colsel, dist)
    return sym
